# baseline (speedup 1.0000x reference)
import functools

import jax
import jax.numpy as jnp
from jax import lax
from jax.experimental import pallas as pl
from jax.experimental.pallas import tpu as pltpu
from jax.experimental.pallas import tpu_sc as plsc

BATCH = 16384
DIM = 128
NC = 2
NS = 16
NW = NC * NS
B_PER_W = BATCH // NW
CHUNK = 128
N_CHUNKS = B_PER_W // CHUNK
UNROLL = 8


def _body(heads_hbm, rels_hbm, tails_hbm, ent_hbm, rel_hbm, out_hbm,
          hidx, ridx, tidx, hrow, rrow, trow, outb, sem):
    wid = lax.axis_index("s") * NC + lax.axis_index("c")
    base = wid * B_PER_W
    lane = lax.iota(jnp.int32, 16)

    for c in range(N_CHUNKS):
        off = base + c * CHUNK
        pltpu.sync_copy(heads_hbm.at[pl.ds(off, CHUNK)], hidx)
        pltpu.sync_copy(rels_hbm.at[pl.ds(off, CHUNK)], ridx)
        pltpu.sync_copy(tails_hbm.at[pl.ds(off, CHUNK)], tidx)
        ch = pltpu.async_copy(ent_hbm.at[hidx], hrow, sem)
        cr = pltpu.async_copy(rel_hbm.at[ridx], rrow, sem)
        ct = pltpu.async_copy(ent_hbm.at[tidx], trow, sem)
        ch.wait()
        cr.wait()
        ct.wait()

        for g in range(CHUNK // 16):
            elem = lane + g * 16

            def step(i, carry):
                acc, jv = carry
                for u in range(UNROLL):
                    ju = jv + u
                    h = plsc.load_gather(hrow, [elem, ju])
                    r = plsc.load_gather(rrow, [elem, ju])
                    t = plsc.load_gather(trow, [elem, ju])
                    acc = acc + jnp.abs(h + r - t)
                return acc, jv + UNROLL

            acc0 = jnp.zeros((16,), jnp.float32)
            jv0 = jnp.zeros((16,), jnp.int32)
            acc, _ = lax.fori_loop(0, DIM // UNROLL, step, (acc0, jv0))
            outb[pl.ds(c * CHUNK + g * 16, 16)] = acc

    pltpu.sync_copy(outb, out_hbm.at[pl.ds(base, B_PER_W)])


@jax.jit
def kernel(heads, rels, tails, ent_embs, rel_embs):
    mesh = plsc.VectorSubcoreMesh(core_axis_name="c", subcore_axis_name="s")
    f = functools.partial(
        pl.kernel,
        mesh=mesh,
        compiler_params=pltpu.CompilerParams(needs_layout_passes=False),
        out_type=jax.ShapeDtypeStruct((BATCH,), jnp.float32),
        scratch_types=[
            pltpu.VMEM((CHUNK,), jnp.int32),
            pltpu.VMEM((CHUNK,), jnp.int32),
            pltpu.VMEM((CHUNK,), jnp.int32),
            pltpu.VMEM((CHUNK, DIM), jnp.float32),
            pltpu.VMEM((CHUNK, DIM), jnp.float32),
            pltpu.VMEM((CHUNK, DIM), jnp.float32),
            pltpu.VMEM((B_PER_W,), jnp.float32),
            pltpu.SemaphoreType.DMA,
        ],
    )(_body)
    return f(heads, rels, tails, ent_embs, rel_embs)


# speedup vs baseline: 3.4719x; 3.4719x over previous
import functools

import jax
import jax.numpy as jnp
from jax import lax
from jax.experimental import pallas as pl
from jax.experimental.pallas import tpu as pltpu
from jax.experimental.pallas import tpu_sc as plsc

BATCH = 16384
DIM = 128
NC = 2
NS = 16
NW = NC * NS
B_PER_W = BATCH // NW
CHUNK = 128
N_CHUNKS = B_PER_W // CHUNK


def _body(heads_hbm, rels_hbm, tails_hbm, ent_hbm, rel_hbm, out_hbm,
          hidx, ridx, tidx, rows, outb, isem, sem0, sem1):
    wid = lax.axis_index("s") * NC + lax.axis_index("c")
    base = wid * B_PER_W
    lane = lax.iota(jnp.int32, 16)
    sems = (sem0, sem1)

    ci = pltpu.async_copy(heads_hbm.at[pl.ds(base, B_PER_W)], hidx, isem)
    cj = pltpu.async_copy(rels_hbm.at[pl.ds(base, B_PER_W)], ridx, isem)
    ck = pltpu.async_copy(tails_hbm.at[pl.ds(base, B_PER_W)], tidx, isem)
    ci.wait()
    cj.wait()
    ck.wait()

    def fire(c):
        buf = c % 2
        s = pl.ds(c * CHUNK, CHUNK)
        return (
            pltpu.async_copy(ent_hbm.at[hidx.at[s]], rows.at[buf, 0], sems[buf]),
            pltpu.async_copy(rel_hbm.at[ridx.at[s]], rows.at[buf, 1], sems[buf]),
            pltpu.async_copy(ent_hbm.at[tidx.at[s]], rows.at[buf, 2], sems[buf]),
        )

    inflight = fire(0)
    for c in range(N_CHUNKS):
        for d in inflight:
            d.wait()
        if c + 1 < N_CHUNKS:
            inflight = fire(c + 1)
        buf = c % 2
        hrow = rows.at[buf, 0]
        rrow = rows.at[buf, 1]
        trow = rows.at[buf, 2]

        def group(g, carry):
            e0 = g * 4
            res = jnp.zeros((16,), jnp.float32)
            for u in range(4):
                e = e0 + u
                acc_a = None
                acc_b = None
                for k in range(DIM // 16):
                    sl = pl.ds(k * 16, 16)
                    term = jnp.abs(hrow[e, sl] + rrow[e, sl] - trow[e, sl])
                    if k % 2 == 0:
                        acc_a = term if acc_a is None else acc_a + term
                    else:
                        acc_b = term if acc_b is None else acc_b + term
                tot = jnp.sum(acc_a + acc_b)
                res = jnp.where(lane == u, tot, res)
            plsc.store_scatter(outb, [c * CHUNK + e0 + lane], res,
                               mask=lane < 4)
            return carry

        lax.fori_loop(0, CHUNK // 4, group, 0)

    pltpu.sync_copy(outb, out_hbm.at[pl.ds(base, B_PER_W)])


@jax.jit
def kernel(heads, rels, tails, ent_embs, rel_embs):
    mesh = plsc.VectorSubcoreMesh(core_axis_name="c", subcore_axis_name="s")
    f = functools.partial(
        pl.kernel,
        mesh=mesh,
        compiler_params=pltpu.CompilerParams(needs_layout_passes=False),
        out_type=jax.ShapeDtypeStruct((BATCH,), jnp.float32),
        scratch_types=[
            pltpu.VMEM((B_PER_W,), jnp.int32),
            pltpu.VMEM((B_PER_W,), jnp.int32),
            pltpu.VMEM((B_PER_W,), jnp.int32),
            pltpu.VMEM((2, 3, CHUNK, DIM), jnp.float32),
            pltpu.VMEM((B_PER_W,), jnp.float32),
            pltpu.SemaphoreType.DMA,
            pltpu.SemaphoreType.DMA,
            pltpu.SemaphoreType.DMA,
        ],
    )(_body)
    return f(heads, rels, tails, ent_embs, rel_embs)
